# baseline (speedup 1.0000x reference)
import functools

import jax
import jax.numpy as jnp
from jax import lax
from jax.experimental import pallas as pl
from jax.experimental.pallas import tpu as pltpu
from jax.experimental.pallas import tpu_sc as plsc

N = 64
H = 512
W = 512
HW = H * W

NUM_CORES = 2
NUM_SUBCORES = 16
NUM_WORKERS = NUM_CORES * NUM_SUBCORES
N_PER_WORKER = N // NUM_WORKERS

CHUNK = 16384
CHUNKS_PER_N = HW // CHUNK
LANES = 16
UNROLL = 8
VEC_ITERS = CHUNK // (LANES * UNROLL)


def _sc_body(frames, ratio_h, dir_h, out, rv, dv, in0, in1, ob):
    wid = lax.axis_index("s") * NUM_CORES + lax.axis_index("c")

    pltpu.sync_copy(ratio_h, rv)
    pltpu.sync_copy(dir_h, dv)

    for j in range(N_PER_WORKER):
        n = wid * N_PER_WORKER + j
        r = rv[n, :]
        d = dv[n, :]
        one = jnp.full((LANES,), 1.0, jnp.float32)
        zero = jnp.full((LANES,), 0.0, jnp.float32)
        av = jnp.where(d > 0, one - r, jnp.where(d < 0, r, one))
        bv = jnp.where(d > 0, r, jnp.where(d < 0, one - r, zero))

        for c in range(CHUNKS_PER_N):
            off = c * CHUNK
            pltpu.sync_copy(frames.at[n, 0, pl.ds(off, CHUNK)], in0)
            pltpu.sync_copy(frames.at[n, 1, pl.ds(off, CHUNK)], in1)

            def blend(i, _):
                base = i * (LANES * UNROLL)
                for u in range(UNROLL):
                    o = base + u * LANES
                    x0 = in0[pl.ds(o, LANES)]
                    x1 = in1[pl.ds(o, LANES)]
                    ob[pl.ds(o, LANES)] = av * x0 + bv * x1
                return 0

            lax.fori_loop(0, VEC_ITERS, blend, 0)
            pltpu.sync_copy(ob, out.at[n, pl.ds(off, CHUNK)])


_sc_call = functools.partial(
    pl.kernel,
    mesh=plsc.VectorSubcoreMesh(core_axis_name="c", subcore_axis_name="s"),
    out_type=jax.ShapeDtypeStruct((N, HW), jnp.float32),
    scratch_types=[
        pltpu.VMEM((N, LANES), jnp.float32),
        pltpu.VMEM((N, LANES), jnp.float32),
        pltpu.VMEM((CHUNK,), jnp.float32),
        pltpu.VMEM((CHUNK,), jnp.float32),
        pltpu.VMEM((CHUNK,), jnp.float32),
    ],
)(_sc_body)


def kernel(exist_frames, ratio, direction):
    frames = exist_frames.reshape(N, 2, HW)
    ratio_b = jnp.broadcast_to(ratio.reshape(N, 1), (N, LANES))
    dir_b = jnp.broadcast_to(direction.reshape(N, 1), (N, LANES))
    out = _sc_call(frames, ratio_b, dir_b)
    return out.reshape(N, 1, H, W)


# speedup vs baseline: 1.0913x; 1.0913x over previous
import functools

import jax
import jax.numpy as jnp
from jax import lax
from jax.experimental import pallas as pl
from jax.experimental.pallas import tpu as pltpu
from jax.experimental.pallas import tpu_sc as plsc

N = 64
H = 512
W = 512
HW = H * W

NUM_CORES = 2
NUM_SUBCORES = 16
NUM_WORKERS = NUM_CORES * NUM_SUBCORES
N_PER_WORKER = N // NUM_WORKERS

CHUNK = 16384
CHUNKS_PER_N = HW // CHUNK
TOTAL_CHUNKS = N_PER_WORKER * CHUNKS_PER_N
LANES = 16
UNROLL = 8


def _sc_body(frames, ratio_h, dir_h, out, rv, dv, in0, in1, ob,
             si0, si1, so0, so1):
    wid = lax.axis_index("s") * NUM_CORES + lax.axis_index("c")

    pltpu.sync_copy(ratio_h, rv)
    pltpu.sync_copy(dir_h, dv)

    one = jnp.full((LANES,), 1.0, jnp.float32)
    zero = jnp.full((LANES,), 0.0, jnp.float32)
    weights = []
    for j in range(N_PER_WORKER):
        n = wid * N_PER_WORKER + j
        r = rv[n, :]
        d = dv[n, :]
        av = jnp.where(d > 0, one - r, jnp.where(d < 0, r, one))
        bv = jnp.where(d > 0, r, jnp.where(d < 0, one - r, zero))
        weights.append((av, bv))

    sems_in = (si0, si1)
    sems_out = (so0, so1)

    def chunk_addr(k):
        n = wid * N_PER_WORKER + (k // CHUNKS_PER_N)
        off = (k % CHUNKS_PER_N) * CHUNK
        return n, off

    def issue_in(k):
        s = k % 2
        n, off = chunk_addr(k)
        h0 = pltpu.async_copy(frames.at[n, 0, pl.ds(off, CHUNK)],
                              in0.at[s], sems_in[s])
        h1 = pltpu.async_copy(frames.at[n, 1, pl.ds(off, CHUNK)],
                              in1.at[s], sems_in[s])
        return h0, h1

    pending_in = issue_in(0)
    pending_out = [None, None]
    for k in range(TOTAL_CHUNKS):
        s = k % 2
        nxt = issue_in(k + 1) if k + 1 < TOTAL_CHUNKS else None
        pending_in[0].wait()
        pending_in[1].wait()
        if pending_out[s] is not None:
            pending_out[s].wait()
        av, bv = weights[k // CHUNKS_PER_N]

        @plsc.parallel_loop(0, CHUNK, step=LANES, unroll=UNROLL)
        def blend(i):
            x0 = in0[s, pl.ds(i, LANES)]
            x1 = in1[s, pl.ds(i, LANES)]
            ob[s, pl.ds(i, LANES)] = av * x0 + bv * x1

        n, off = chunk_addr(k)
        pending_out[s] = pltpu.async_copy(
            ob.at[s], out.at[n, pl.ds(off, CHUNK)], sems_out[s])
        pending_in = nxt
    pending_out[0].wait()
    pending_out[1].wait()


_sc_call = functools.partial(
    pl.kernel,
    mesh=plsc.VectorSubcoreMesh(core_axis_name="c", subcore_axis_name="s"),
    out_type=jax.ShapeDtypeStruct((N, HW), jnp.float32),
    scratch_types=[
        pltpu.VMEM((N, LANES), jnp.float32),
        pltpu.VMEM((N, LANES), jnp.float32),
        pltpu.VMEM((2, CHUNK), jnp.float32),
        pltpu.VMEM((2, CHUNK), jnp.float32),
        pltpu.VMEM((2, CHUNK), jnp.float32),
        pltpu.SemaphoreType.DMA,
        pltpu.SemaphoreType.DMA,
        pltpu.SemaphoreType.DMA,
        pltpu.SemaphoreType.DMA,
    ],
)(_sc_body)


def kernel(exist_frames, ratio, direction):
    frames = exist_frames.reshape(N, 2, HW)
    ratio_b = jnp.broadcast_to(ratio.reshape(N, 1), (N, LANES))
    dir_b = jnp.broadcast_to(direction.reshape(N, 1), (N, LANES))
    out = _sc_call(frames, ratio_b, dir_b)
    return out.reshape(N, 1, H, W)


# speedup vs baseline: 5.2964x; 4.8533x over previous
import functools

import jax
import jax.numpy as jnp
from jax import lax
from jax.experimental import pallas as pl
from jax.experimental.pallas import tpu as pltpu
from jax.experimental.pallas import tpu_sc as plsc

N = 64
H = 512
W = 512

NUM_CORES = 2
NUM_SUBCORES = 16
NUM_WORKERS = NUM_CORES * NUM_SUBCORES
N_PER_WORKER = N // NUM_WORKERS

ROWS = 32
CHUNKS_PER_N = H // ROWS
TOTAL_CHUNKS = N_PER_WORKER * CHUNKS_PER_N
LANES = 16
SEGS = W // LANES
VECS = ROWS * SEGS
UNROLL = 8


def _sc_body(frames, ratio_h, dir_h, out, rv, dv, in0, in1, ob,
             si0, si1, so0, so1):
    wid = lax.axis_index("s") * NUM_CORES + lax.axis_index("c")

    pltpu.sync_copy(ratio_h, rv)
    pltpu.sync_copy(dir_h, dv)

    one = jnp.full((LANES,), 1.0, jnp.float32)
    zero = jnp.full((LANES,), 0.0, jnp.float32)
    weights = []
    for j in range(N_PER_WORKER):
        n = wid * N_PER_WORKER + j
        r = rv[n, :]
        d = dv[n, :]
        av = jnp.where(d > 0, one - r, jnp.where(d < 0, r, one))
        bv = jnp.where(d > 0, r, jnp.where(d < 0, one - r, zero))
        weights.append((av, bv))

    sems_in = (si0, si1)
    sems_out = (so0, so1)

    def chunk_addr(k):
        n = wid * N_PER_WORKER + (k // CHUNKS_PER_N)
        row0 = (k % CHUNKS_PER_N) * ROWS
        return n, row0

    def issue_in(k):
        s = k % 2
        n, row0 = chunk_addr(k)
        h0 = pltpu.async_copy(frames.at[n, 0, pl.ds(row0, ROWS), :],
                              in0.at[s], sems_in[s])
        h1 = pltpu.async_copy(frames.at[n, 1, pl.ds(row0, ROWS), :],
                              in1.at[s], sems_in[s])
        return h0, h1

    pending_in = issue_in(0)
    pending_out = [None, None]
    for k in range(TOTAL_CHUNKS):
        s = k % 2
        nxt = issue_in(k + 1) if k + 1 < TOTAL_CHUNKS else None
        for h in pending_in:
            h.wait()
        if pending_out[s] is not None:
            pending_out[s].wait()
        av, bv = weights[k // CHUNKS_PER_N]

        @plsc.parallel_loop(0, VECS, step=1, unroll=UNROLL)
        def blend(i):
            r = i // SEGS
            c = (i % SEGS) * LANES
            x0 = in0[s, r, pl.ds(c, LANES)]
            x1 = in1[s, r, pl.ds(c, LANES)]
            ob[s, r, pl.ds(c, LANES)] = av * x0 + bv * x1

        n, row0 = chunk_addr(k)
        pending_out[s] = pltpu.async_copy(
            ob.at[s], out.at[n, 0, pl.ds(row0, ROWS), :], sems_out[s])
        pending_in = nxt
    pending_out[0].wait()
    pending_out[1].wait()


_sc_call = functools.partial(
    pl.kernel,
    mesh=plsc.VectorSubcoreMesh(core_axis_name="c", subcore_axis_name="s"),
    out_type=jax.ShapeDtypeStruct((N, 1, H, W), jnp.float32),
    compiler_params=pltpu.CompilerParams(use_tc_tiling_on_sc=True),
    scratch_types=[
        pltpu.VMEM((N, LANES), jnp.float32),
        pltpu.VMEM((N, LANES), jnp.float32),
        pltpu.VMEM((2, ROWS, W), jnp.float32),
        pltpu.VMEM((2, ROWS, W), jnp.float32),
        pltpu.VMEM((2, ROWS, W), jnp.float32),
        pltpu.SemaphoreType.DMA,
        pltpu.SemaphoreType.DMA,
        pltpu.SemaphoreType.DMA,
        pltpu.SemaphoreType.DMA,
    ],
)(_sc_body)


def kernel(exist_frames, ratio, direction):
    ratio_b = jnp.broadcast_to(ratio.reshape(N, 1), (N, LANES))
    dir_b = jnp.broadcast_to(direction.reshape(N, 1), (N, LANES))
    return _sc_call(exist_frames, ratio_b, dir_b)


# speedup vs baseline: 5.3139x; 1.0033x over previous
import functools

import jax
import jax.numpy as jnp
from jax import lax
from jax.experimental import pallas as pl
from jax.experimental.pallas import tpu as pltpu
from jax.experimental.pallas import tpu_sc as plsc

N = 64
H = 512
W = 512

NUM_CORES = 2
NUM_SUBCORES = 16
NUM_WORKERS = NUM_CORES * NUM_SUBCORES
N_PER_WORKER = N // NUM_WORKERS

ROWS = 32
CHUNKS_PER_N = H // ROWS
TOTAL_CHUNKS = N_PER_WORKER * CHUNKS_PER_N
SLOTS = 3
LANES = 16
SEGS = W // LANES
VECS = ROWS * SEGS
UNROLL = 8


def _sc_body(frames, ratio_h, dir_h, out, rv, dv, in0, in1,
             sin, sout):
    wid = lax.axis_index("s") * NUM_CORES + lax.axis_index("c")

    pltpu.sync_copy(ratio_h, rv)
    pltpu.sync_copy(dir_h, dv)

    one = jnp.full((LANES,), 1.0, jnp.float32)
    zero = jnp.full((LANES,), 0.0, jnp.float32)
    weights = []
    for j in range(N_PER_WORKER):
        n = wid * N_PER_WORKER + j
        r = rv[n, :]
        d = dv[n, :]
        av = jnp.where(d > 0, one - r, jnp.where(d < 0, r, one))
        bv = jnp.where(d > 0, r, jnp.where(d < 0, one - r, zero))
        weights.append((av, bv))

    def chunk_addr(k):
        n = wid * N_PER_WORKER + (k // CHUNKS_PER_N)
        row0 = (k % CHUNKS_PER_N) * ROWS
        return n, row0

    def issue_in(k):
        s = k % SLOTS
        n, row0 = chunk_addr(k)
        h0 = pltpu.async_copy(frames.at[n, 0, pl.ds(row0, ROWS), :],
                              in0.at[s], sin[s])
        h1 = pltpu.async_copy(frames.at[n, 1, pl.ds(row0, ROWS), :],
                              in1.at[s], sin[s])
        return h0, h1

    pending_in = [None] * SLOTS
    pending_out = [None] * SLOTS
    for k in range(SLOTS - 1):
        pending_in[k % SLOTS] = issue_in(k)

    for k in range(TOTAL_CHUNKS):
        s = k % SLOTS
        for h in pending_in[s]:
            h.wait()
        av, bv = weights[k // CHUNKS_PER_N]

        @plsc.parallel_loop(0, VECS, step=1, unroll=UNROLL)
        def blend(i):
            r = i // SEGS
            c = (i % SEGS) * LANES
            x0 = in0[s, r, pl.ds(c, LANES)]
            x1 = in1[s, r, pl.ds(c, LANES)]
            in0[s, r, pl.ds(c, LANES)] = av * x0 + bv * x1

        n, row0 = chunk_addr(k)
        pending_out[s] = pltpu.async_copy(
            in0.at[s], out.at[n, 0, pl.ds(row0, ROWS), :], sout[s])
        nk = k + SLOTS - 1
        if nk < TOTAL_CHUNKS:
            ns = nk % SLOTS
            if pending_out[ns] is not None:
                pending_out[ns].wait()
            pending_in[ns] = issue_in(nk)
    for s in range(SLOTS):
        if pending_out[s] is not None:
            pending_out[s].wait()


_sc_call = functools.partial(
    pl.kernel,
    mesh=plsc.VectorSubcoreMesh(core_axis_name="c", subcore_axis_name="s"),
    out_type=jax.ShapeDtypeStruct((N, 1, H, W), jnp.float32),
    compiler_params=pltpu.CompilerParams(use_tc_tiling_on_sc=True),
    scratch_types=[
        pltpu.VMEM((N, LANES), jnp.float32),
        pltpu.VMEM((N, LANES), jnp.float32),
        pltpu.VMEM((SLOTS, ROWS, W), jnp.float32),
        pltpu.VMEM((SLOTS, ROWS, W), jnp.float32),
        [pltpu.SemaphoreType.DMA] * SLOTS,
        [pltpu.SemaphoreType.DMA] * SLOTS,
    ],
)(_sc_body)


def kernel(exist_frames, ratio, direction):
    ratio_b = jnp.broadcast_to(ratio.reshape(N, 1), (N, LANES))
    dir_b = jnp.broadcast_to(direction.reshape(N, 1), (N, LANES))
    return _sc_call(exist_frames, ratio_b, dir_b)
